# baseline (speedup 1.0000x reference)
import functools

import jax
import jax.numpy as jnp
from jax.experimental import pallas as pl

_ROW_LANES = 0
_ROW_BM = 640
_ROW_BI = 641
_ROW_BH = 644
_ROW_NPX = 647
_ROW_NPY = 648
_ROW_SE = 649
_ROW_VA = 650
_ROWS = 651


def _dn(a, b):
    return jax.lax.dot_general(a, b, (((1,), (1,)), ((), ())),
                               preferred_element_type=jnp.float32)


def _body(B, P, S, H, N,
          mp_ref, actx_ref, nctx_tab_ref, Wm_ref, Wih_ref, Whh_ref, out_ref):
    BP = B * P
    npx_tab = mp_ref[_ROW_NPX].reshape(1, N)
    npy_tab = mp_ref[_ROW_NPY].reshape(1, N)
    se_row = mp_ref[_ROW_SE]
    starts = se_row[0:B].reshape(1, B)
    ends = se_row[B:2 * B].reshape(1, B)
    valid = mp_ref[_ROW_VA][0:B].reshape(1, B)

    iota_n = jax.lax.broadcasted_iota(jnp.int32, (N, B), 0).astype(jnp.float32)
    onehotT = (iota_n == starts).astype(jnp.float32)

    nctx = jax.lax.dot_general(onehotT, nctx_tab_ref[...],
                               (((0,), (0,)), ((), ())),
                               preferred_element_type=jnp.float32)
    npx_row = jnp.dot(npx_tab, onehotT,
                      preferred_element_type=jnp.float32)
    npy_row = jnp.dot(npy_tab, onehotT,
                      preferred_element_type=jnp.float32)
    npx = jnp.transpose(npx_row)
    npy = jnp.transpose(npy_row)
    npxy = jnp.concatenate([npx, npy], axis=1)

    actx = actx_ref[...]
    Wm = Wm_ref[...]
    xg = (_dn(nctx, Wm[:, 2:2 + H]) + _dn(actx, Wm[:, 2 + H:])
          + _dn(-npxy, Wm[:, 0:2]) + mp_ref[_ROW_BM])
    x = jnp.maximum(xg, 0.0)

    gi = _dn(x, Wih_ref[...])
    gh = _dn(nctx, Whh_ref[...])
    r_g = jax.nn.sigmoid(gi[:, :H] + mp_ref[_ROW_BI]
                         + gh[:, :H] + mp_ref[_ROW_BH])
    z = jax.nn.sigmoid(gi[:, H:2 * H] + mp_ref[_ROW_BI + 1]
                       + gh[:, H:2 * H] + mp_ref[_ROW_BH + 1])
    n_g = jnp.tanh(gi[:, 2 * H:] + mp_ref[_ROW_BI + 2]
                   + r_g * (gh[:, 2 * H:] + mp_ref[_ROW_BH + 2]))
    r = (1.0 - z) * n_g + z * nctx

    condf = jnp.where(
        jnp.logical_and(valid > 0.0, (ends - starts) > 0.0), 1.0, 0.0)
    cond_col = jnp.transpose(condf)

    iota_p = jax.lax.broadcasted_iota(jnp.int32, (BP, B), 0)
    iota_b = jax.lax.broadcasted_iota(jnp.int32, (BP, B), 1)
    rep = jnp.logical_and(iota_p >= P * iota_b,
                          iota_p < P * iota_b + P).astype(jnp.float32)

    np640 = jnp.dot(rep, npxy, preferred_element_type=jnp.float32)
    npx6 = np640[:, 0:1]
    npy6 = np640[:, 1:2]
    cond6 = jnp.dot(rep, cond_col, preferred_element_type=jnp.float32)

    lanes = mp_ref[_ROW_LANES:_ROW_LANES + BP, :]
    iota_k = jax.lax.broadcasted_iota(jnp.int32, (1, 128), 1)
    even = (iota_k % 2 == 0) & (iota_k < 2 * S)
    npil = jnp.where(iota_k % 2 == 0, npx6, npy6)
    diff = lanes - npil
    sq = diff * diff
    sq_shift = jnp.concatenate([sq[:, 1:], jnp.zeros((BP, 1), jnp.float32)],
                               axis=1)
    sum2 = jnp.where(even, sq + sq_shift, jnp.inf)
    d2min = jnp.min(sum2, axis=1, keepdims=True)
    real = iota_k < 2 * S
    nan_any = jnp.any(jnp.isnan(lanes) & real, axis=1, keepdims=True)
    d2 = jnp.where(nan_any, npx6 * npx6 + npy6 * npy6, d2min)
    keep = jnp.logical_and(cond6 > 0.5, d2 < 10000.0)

    r_rep = jnp.dot(rep, r, preferred_element_type=jnp.float32)
    out_ref[...] = jnp.where(keep, r_rep, 0.0)


def kernel(agent_pos, agent_context, ngh_pos, ngh_context, possible_lanes,
           lane_context, label, seq_start_end, valid_neighbor,
           W_msg, b_msg, W_ih, W_hh, b_ih, b_hh):
    B, P, H = lane_context.shape
    S = possible_lanes.shape[0]
    N = ngh_context.shape[0]

    lanes = jnp.pad(possible_lanes.transpose(1, 0, 2).reshape(B * P, 2 * S),
                    ((0, 0), (0, 128 - 2 * S)), constant_values=1e30)
    se = jnp.concatenate([seq_start_end[:, 0], seq_start_end[:, 1]]
                         ).astype(jnp.float32).reshape(1, 128)
    va = jnp.pad(valid_neighbor.astype(jnp.float32), (0, 128 - B)
                 ).reshape(1, 128)
    mp = jnp.concatenate([
        lanes,
        b_msg.reshape(1, 128),
        b_ih.reshape(3, 128),
        b_hh.reshape(3, 128),
        ngh_pos[:, 0].reshape(1, 128),
        ngh_pos[:, 1].reshape(1, 128),
        se,
        va,
    ], axis=0)

    body = functools.partial(_body, B, P, S, H, N)
    out2 = pl.pallas_call(
        body,
        out_shape=jax.ShapeDtypeStruct((B * P, H), jnp.float32),
    )(mp, agent_context, ngh_context, W_msg, W_ih, W_hh)

    return (lane_context, out2.reshape(B, P, H))


# speedup vs baseline: 1.0436x; 1.0436x over previous
import functools

import jax
import jax.numpy as jnp
from jax.experimental import pallas as pl

_OFF_BM = 0
_OFF_BI = 128
_OFF_BH = 512
_OFF_NPX = 896
_OFF_NPY = 1024
_OFF_ST = 1152
_OFF_EN = 1216
_OFF_VA = 1280
_PACK_LEN = 1344


def _dn(a, b):
    return jax.lax.dot_general(a, b, (((1,), (1,)), ((), ())),
                               preferred_element_type=jnp.float32)


def _body(B, P, S, H, N,
          pack_ref, actx_ref, nctx_tab_ref, lanes_ref,
          Wm_ref, Wih_ref, Whh_ref, out_ref):
    BP = B * P
    pack = pack_ref[...]
    bm = pack[_OFF_BM:_OFF_BM + H]
    bi = pack[_OFF_BI:_OFF_BI + 3 * H]
    bh = pack[_OFF_BH:_OFF_BH + 3 * H]
    npx_tab = pack[_OFF_NPX:_OFF_NPX + N].reshape(1, N)
    npy_tab = pack[_OFF_NPY:_OFF_NPY + N].reshape(1, N)
    starts = pack[_OFF_ST:_OFF_ST + B].reshape(1, B)
    ends = pack[_OFF_EN:_OFF_EN + B].reshape(1, B)
    valid = pack[_OFF_VA:_OFF_VA + B].reshape(1, B)

    iota_n = jax.lax.broadcasted_iota(jnp.int32, (N, B), 0).astype(jnp.float32)
    onehotT = (iota_n == starts).astype(jnp.float32)

    nctx = jax.lax.dot_general(onehotT, nctx_tab_ref[...],
                               (((0,), (0,)), ((), ())),
                               preferred_element_type=jnp.float32)
    npx_row = jnp.dot(npx_tab, onehotT,
                      preferred_element_type=jnp.float32)
    npy_row = jnp.dot(npy_tab, onehotT,
                      preferred_element_type=jnp.float32)
    npx = jnp.transpose(npx_row)
    npy = jnp.transpose(npy_row)
    npxy = jnp.concatenate([npx, npy], axis=1)

    actx = actx_ref[...]
    Wm = Wm_ref[...]
    xg = (_dn(nctx, Wm[:, 2:2 + H]) + _dn(actx, Wm[:, 2 + H:])
          + _dn(-npxy, Wm[:, 0:2]) + bm)
    x = jnp.maximum(xg, 0.0)

    gi = _dn(x, Wih_ref[...]) + bi
    gh = _dn(nctx, Whh_ref[...]) + bh
    r_g = jax.nn.sigmoid(gi[:, :H] + gh[:, :H])
    z = jax.nn.sigmoid(gi[:, H:2 * H] + gh[:, H:2 * H])
    n_g = jnp.tanh(gi[:, 2 * H:] + r_g * gh[:, 2 * H:])
    r = (1.0 - z) * n_g + z * nctx

    condf = jnp.where(
        jnp.logical_and(valid > 0.0, (ends - starts) > 0.0), 1.0, 0.0)
    cond_col = jnp.transpose(condf)

    iota_p = jax.lax.broadcasted_iota(jnp.int32, (BP, B), 0)
    iota_b = jax.lax.broadcasted_iota(jnp.int32, (BP, B), 1)
    rep = jnp.logical_and(iota_p >= P * iota_b,
                          iota_p < P * iota_b + P).astype(jnp.float32)

    np640 = jnp.dot(rep, npxy, preferred_element_type=jnp.float32)
    npx6 = np640[:, 0:1]
    npy6 = np640[:, 1:2]
    cond6 = jnp.dot(rep, cond_col, preferred_element_type=jnp.float32)

    lanes = lanes_ref[...]
    iota_k = jax.lax.broadcasted_iota(jnp.int32, (1, 2 * S), 1)
    even = (iota_k % 2 == 0)
    npil = jnp.where(even, npx6, npy6)
    diff = lanes - npil
    sq = diff * diff
    sq_shift = jnp.concatenate([sq[:, 1:], jnp.zeros((BP, 1), jnp.float32)],
                               axis=1)
    sum2 = jnp.where(even, sq + sq_shift, jnp.inf)
    d2min = jnp.min(sum2, axis=1, keepdims=True)
    nan_any = jnp.any(jnp.isnan(lanes), axis=1, keepdims=True)
    d2 = jnp.where(nan_any, npx6 * npx6 + npy6 * npy6, d2min)
    keep = jnp.logical_and(cond6 > 0.5, d2 < 10000.0)

    r_rep = jnp.dot(rep, r, preferred_element_type=jnp.float32)
    out_ref[...] = jnp.where(keep, r_rep, 0.0).reshape(B, P, H)


def kernel(agent_pos, agent_context, ngh_pos, ngh_context, possible_lanes,
           lane_context, label, seq_start_end, valid_neighbor,
           W_msg, b_msg, W_ih, W_hh, b_ih, b_hh):
    B, P, H = lane_context.shape
    S = possible_lanes.shape[0]
    N = ngh_context.shape[0]

    pack = jnp.concatenate([
        b_msg, b_ih, b_hh,
        ngh_pos[:, 0], ngh_pos[:, 1],
        seq_start_end[:, 0].astype(jnp.float32),
        seq_start_end[:, 1].astype(jnp.float32),
        valid_neighbor.astype(jnp.float32),
    ])
    lanes = possible_lanes.transpose(1, 0, 2).reshape(B * P, 2 * S)

    body = functools.partial(_body, B, P, S, H, N)
    out2 = pl.pallas_call(
        body,
        out_shape=jax.ShapeDtypeStruct((B, P, H), jnp.float32),
    )(pack, agent_context, ngh_context, lanes, W_msg, W_ih, W_hh)

    return (lane_context, out2)
